# baseline (speedup 1.0000x reference)
import functools

import numpy as np
import jax
import jax.numpy as jnp
from jax import lax
from jax.experimental import pallas as pl
from jax.experimental.pallas import tpu as pltpu
from jax.experimental.pallas import tpu_sc as plsc

IMG = 224
PATCH = 14
NPATCH = (IMG // PATCH) ** 2
PDIM = PATCH * PATCH
PDIM_PAD = 256
D = 512
E = 8
NHEAD = 8
HD = D // NHEAD
FF = 4 * D
B = 8
NTOK = B * NPATCH
EPAD = 128
NEG = -1e30

R = 256
G = (2 * NTOK) // R + E
S = G * R


def _mm(a, b):
    return lax.dot_general(a, b, (((a.ndim - 1,), (0,)), ((), ())),
                           preferred_element_type=jnp.float32)


def _mm_t(a, b):
    return lax.dot_general(a, b, (((1,), (1,)), ((), ())),
                           preferred_element_type=jnp.float32)


def _pos_encoding_np():
    pos = np.arange(NPATCH, dtype=np.float32)[:, None]
    div = np.exp(np.arange(0, D, 2, dtype=np.float32)
                 * (-np.log(10000.0) / D)).astype(np.float32)
    pe = np.zeros((NPATCH, D), np.float32)
    pe[:, 0::2] = np.sin(pos * div)
    pe[:, 1::2] = np.cos(pos * div)
    return pe


def _noise_np(seed):
    n = np.asarray(jax.random.normal(jax.random.key(seed), (B, NPATCH, E),
                                     dtype=jnp.float32)).reshape(NTOK, E)
    return np.pad(n, ((0, 0), (0, EPAD - E)))


_PE = _pos_encoding_np()
_NOISE1 = _noise_np(1)
_NOISE2 = _noise_np(2)


def _row(v):
    return v.reshape(1, -1)


def _pad_e(w):
    return jnp.pad(w, [(0, 0)] * (w.ndim - 1) + [(0, EPAD - E)])


def _mega_body(embed, refs):
    if embed:
        (xin_ref, wpe_ref, bpe_ref, pe_ref,
         wip_ref, bip_ref, wq_ref, wk_ref, wv_ref, wo_ref, bo_ref,
         wr_ref, br_ref, wn_ref, bn_ref, noise_ref,
         attn_ref, route_ref, posi0_ref, posi1_ref, spi_ref,
         route_scr, carry, carr_all, offs) = refs
    else:
        (y0_ref, y1_ref, routep_ref, lng_ref, lnb_ref, wvp_ref, bvp_ref,
         wip_ref, bip_ref, wq_ref, wk_ref, wv_ref, wo_ref, bo_ref,
         wr_ref, br_ref, wn_ref, bn_ref, noise_ref,
         fv_ref, attn_ref, route_ref, posi0_ref, posi1_ref, spi_ref,
         route_scr, carry, carr_all, offs) = refs
    g = pl.program_id(0)
    colid = lax.broadcasted_iota(jnp.int32, (NPATCH, EPAD), 1)
    colf = colid.astype(jnp.float32)

    @pl.when(g < B)
    def _():
        if embed:
            x = _mm(xin_ref[...], wpe_ref[...]) + bpe_ref[...] + pe_ref[...]
        else:
            blkp = routep_ref[...]
            g0 = jnp.sum(jnp.where(colid == 2, blkp, 0.0), axis=1,
                         keepdims=True)
            g1 = jnp.sum(jnp.where(colid == 3, blkp, 0.0), axis=1,
                         keepdims=True)
            a = g0 * y0_ref[...] + g1 * y1_ref[...]
            mu = jnp.mean(a, axis=1, keepdims=True)
            var = jnp.mean((a - mu) ** 2, axis=1, keepdims=True)
            ln = (lng_ref[...] * (a - mu) * lax.rsqrt(var + 1e-5)
                  + lnb_ref[...])
            x = _mm(ln, wvp_ref[...]) + bvp_ref[...]
            fv_ref[...] = x
        y = _mm(x, wip_ref[...]) + bip_ref[...]
        q = _mm(y, wq_ref[...])
        k = _mm(y, wk_ref[...])
        v = _mm(y, wv_ref[...])
        row = lax.broadcasted_iota(jnp.int32, (NPATCH, NPATCH), 0)
        colt = lax.broadcasted_iota(jnp.int32, (NPATCH, NPATCH), 1)
        causal = colt <= row
        o_parts = []
        for h in range(NHEAD):
            s = h * HD
            qh = q[:, s:s + HD]
            kh = k[:, s:s + HD]
            vh = v[:, s:s + HD]
            wei = _mm_t(qh, kh) * (HD ** -0.5)
            wei = jnp.where(causal, wei, NEG)
            m = jnp.max(wei, axis=1, keepdims=True)
            p = jnp.exp(wei - m)
            p = p / jnp.sum(p, axis=1, keepdims=True)
            o_parts.append(_mm(p, vh))
        o = jnp.concatenate(o_parts, axis=1)
        attn = _mm(o, wo_ref[...]) + bo_ref[...]
        attn_ref[...] = attn

        logits = _mm(attn, wr_ref[...]) + br_ref[...]
        nl = _mm(attn, wn_ref[...]) + bn_ref[...]
        sp = jnp.maximum(nl, 0.0) + jnp.log1p(jnp.exp(-jnp.abs(nl)))
        noisy = logits + noise_ref[...] * sp
        noisy = jnp.where(colid < E, noisy, NEG)
        m1 = jnp.max(noisy, axis=1, keepdims=True)
        e0 = jnp.min(jnp.where(noisy == m1, colid, EPAD), axis=1,
                     keepdims=True)
        m2 = jnp.max(jnp.where(colid == e0, NEG, noisy), axis=1,
                     keepdims=True)
        e1 = jnp.min(jnp.where((noisy == m2) & (colid != e0), colid, EPAD),
                     axis=1, keepdims=True)
        selp = jnp.where(noisy >= m2, jnp.exp(noisy - m1), 0.0)
        z = jnp.sum(selp, axis=1, keepdims=True)
        g0o = 1.0 / z
        g1o = jnp.exp(m2 - m1) / z
        route = (jnp.where(colid == 0, e0.astype(jnp.float32), 0.0)
                 + jnp.where(colid == 1, e1.astype(jnp.float32), 0.0)
                 + jnp.where(colid == 2, g0o, 0.0)
                 + jnp.where(colid == 3, g1o, 0.0))
        route_ref[...] = route
        route_scr[pl.ds(g * NPATCH, NPATCH), :] = route

        oh0 = jnp.where(colf == e0.astype(jnp.float32), 1.0, 0.0)
        oh1 = jnp.where(colf == e1.astype(jnp.float32), 1.0, 0.0)

        @pl.when(g == 0)
        def _():
            carry[...] = jnp.zeros_like(carry)

        carr_all[pl.ds(g, 1), :] = carry[...]
        carry[...] += (jnp.sum(oh0, axis=0, keepdims=True)
                       + jnp.sum(oh1, axis=0, keepdims=True))

    @pl.when(g == B)
    def _():
        counts = carry[...]
        pcr = jnp.ceil(counts / R)
        rowi = lax.broadcasted_iota(jnp.int32, (EPAD, EPAD), 0)
        coli = lax.broadcasted_iota(jnp.int32, (EPAD, EPAD), 1)
        incl = jnp.where(rowi <= coli, 1.0, 0.0)
        strict = jnp.where(rowi < coli, 1.0, 0.0)
        ends = _mm(pcr, incl)
        offs[...] = _mm(pcr * R, strict)
        ge = jnp.where((coli < E) & (rowi.astype(jnp.float32)
                                     >= jnp.broadcast_to(ends, (EPAD, EPAD))),
                       1.0, 0.0)
        texp = jnp.minimum(jnp.sum(ge, axis=1, keepdims=True),
                           float(E - 1))
        nt = jnp.sum(jnp.where(coli[0:1] == E - 1, ends, 0.0), axis=1,
                     keepdims=True)
        rowc = lax.broadcasted_iota(jnp.int32, (EPAD, 1), 0)
        spi_ref[...] = jnp.where(rowc == G, nt, texp).astype(jnp.int32)

    @pl.when(g >= B)
    def _():
        t = g - B
        blk = route_scr[pl.ds(t * NPATCH, NPATCH), :]
        oh0 = jnp.where(colf == blk[:, 0:1], 1.0, 0.0)
        oh1 = jnp.where(colf == blk[:, 1:2], 1.0, 0.0)
        carr = carr_all[pl.ds(t, 1), :]
        rowi = lax.broadcasted_iota(jnp.int32, (NPATCH, NPATCH), 0)
        coli = lax.broadcasted_iota(jnp.int32, (NPATCH, NPATCH), 1)
        tri = jnp.where(rowi >= coli, 1.0, 0.0)
        c0 = _mm(tri, oh0) - oh0
        s0 = jnp.sum(oh0, axis=0, keepdims=True)
        c1 = _mm(tri, oh1) - oh1 + s0
        base = offs[...] + carr
        posi0_ref[...] = jnp.sum(oh0 * (base + c0), axis=1,
                                 keepdims=True).astype(jnp.int32)
        posi1_ref[...] = jnp.sum(oh1 * (base + c1), axis=1,
                                 keepdims=True).astype(jnp.int32)


def _mega(p, noise, embed, front_args):
    img = lambda bs: pl.BlockSpec(bs, lambda g: (jnp.minimum(g, B - 1), 0))
    full = lambda a: pl.BlockSpec(a.shape, lambda g: (0,) * a.ndim)
    wr = _pad_e(p['Wr'])
    br = _row(_pad_e(p['br']))
    wn = _pad_e(p['Wn'])
    bn = _row(_pad_e(p['bn']))
    shared = [p['Wip'], _row(p['bip']), p['Wq'], p['Wk'], p['Wv'], p['Wo'],
              _row(p['bo']), wr, br, wn, bn]
    if embed:
        xin, wpe, bpe, pe = front_args
        args = [xin, wpe, _row(bpe), pe] + shared + [noise]
        in_specs = ([img((NPATCH, PDIM_PAD))] + [full(a) for a in args[1:4]]
                    + [full(a) for a in shared]
                    + [img((NPATCH, EPAD))])
        n_extra_out = 0
    else:
        y0, y1, routep, lng, lnb, wvp, bvp = front_args
        args = ([y0, y1, routep, _row(lng), _row(lnb), wvp, _row(bvp)]
                + shared + [noise])
        in_specs = ([img((NPATCH, D)), img((NPATCH, D)), img((NPATCH, EPAD))]
                    + [full(a) for a in args[3:7]]
                    + [full(a) for a in shared]
                    + [img((NPATCH, EPAD))])
        n_extra_out = 1

    out_specs = [img((NPATCH, D)), img((NPATCH, EPAD)),
                 pl.BlockSpec((NPATCH, 1), lambda g: (jnp.maximum(g - B, 0), 0)),
                 pl.BlockSpec((NPATCH, 1), lambda g: (jnp.maximum(g - B, 0), 0)),
                 pl.BlockSpec((EPAD, 1), lambda g: (0, 0))]
    out_shape = [jax.ShapeDtypeStruct((NTOK, D), jnp.float32),
                 jax.ShapeDtypeStruct((NTOK, EPAD), jnp.float32),
                 jax.ShapeDtypeStruct((NTOK, 1), jnp.int32),
                 jax.ShapeDtypeStruct((NTOK, 1), jnp.int32),
                 jax.ShapeDtypeStruct((EPAD, 1), jnp.int32)]
    if not embed:
        out_specs = [img((NPATCH, D))] + out_specs
        out_shape = [jax.ShapeDtypeStruct((NTOK, D), jnp.float32)] + out_shape

    def body(*refs):
        _mega_body(embed, refs)

    outs = pl.pallas_call(
        body,
        grid=(2 * B,),
        in_specs=in_specs,
        out_specs=out_specs,
        out_shape=out_shape,
        scratch_shapes=[pltpu.VMEM((NTOK, EPAD), jnp.float32),
                        pltpu.VMEM((1, EPAD), jnp.float32),
                        pltpu.VMEM((B, EPAD), jnp.float32),
                        pltpu.VMEM((1, EPAD), jnp.float32)],
    )(*args)
    return outs


_NW = 32
_CH = NTOK // _NW


def _wid():
    return lax.axis_index("s") * 2 + lax.axis_index("c")


@functools.cache
def _sc_dispatch_kernel():
    mesh = plsc.VectorSubcoreMesh(core_axis_name="c", subcore_axis_name="s")

    @functools.partial(
        pl.kernel, mesh=mesh,
        out_type=jax.ShapeDtypeStruct((S, D), jnp.float32),
        scratch_types=[pltpu.VMEM((_CH,), jnp.int32),
                       pltpu.VMEM((_CH, D), jnp.float32),
                       pltpu.SemaphoreType.DMA],
    )
    def k(attn_hbm, pos0_hbm, pos1_hbm, xg_hbm, idx_v, rows_v, sem):
        base = _wid() * _CH
        pltpu.sync_copy(attn_hbm.at[pl.ds(base, _CH)], rows_v)
        pltpu.sync_copy(pos0_hbm.at[pl.ds(base, _CH)], idx_v)
        pltpu.async_copy(rows_v, xg_hbm.at[idx_v], sem).wait()
        pltpu.sync_copy(pos1_hbm.at[pl.ds(base, _CH)], idx_v)
        pltpu.async_copy(rows_v, xg_hbm.at[idx_v], sem).wait()

    return k


def _sc_dispatch(attn, pos0, pos1):
    return _sc_dispatch_kernel()(attn, pos0, pos1)


@functools.cache
def _sc_combine_kernel():
    mesh = plsc.VectorSubcoreMesh(core_axis_name="c", subcore_axis_name="s")

    @functools.partial(
        pl.kernel, mesh=mesh,
        out_type=[jax.ShapeDtypeStruct((NTOK, D), jnp.float32),
                  jax.ShapeDtypeStruct((NTOK, D), jnp.float32)],
        scratch_types=[pltpu.VMEM((_CH,), jnp.int32),
                       pltpu.VMEM((_CH, D), jnp.float32),
                       pltpu.SemaphoreType.DMA],
    )
    def k(eo_hbm, pos0_hbm, pos1_hbm, y0_hbm, y1_hbm, idx_v, rows_v, sem):
        base = _wid() * _CH
        pltpu.sync_copy(pos0_hbm.at[pl.ds(base, _CH)], idx_v)
        pltpu.async_copy(eo_hbm.at[idx_v], rows_v, sem).wait()
        pltpu.sync_copy(rows_v, y0_hbm.at[pl.ds(base, _CH)])
        pltpu.sync_copy(pos1_hbm.at[pl.ds(base, _CH)], idx_v)
        pltpu.async_copy(eo_hbm.at[idx_v], rows_v, sem).wait()
        pltpu.sync_copy(rows_v, y1_hbm.at[pl.ds(base, _CH)])

    return k


def _sc_combine(eo, pos0, pos1):
    return _sc_combine_kernel()(eo, pos0, pos1)


def _gexpert_body(sp_ref, xg_ref, w1_ref, b1_ref, w2_ref, b2_ref, eo_ref):
    g = pl.program_id(0)

    @pl.when(g < sp_ref[G])
    def _():
        h = jnp.maximum(_mm(xg_ref[...], w1_ref[0]) + b1_ref[0], 0.0)
        eo_ref[...] = _mm(h, w2_ref[0]) + b2_ref[0]


def _gexpert(sp, xg, p):
    grid_spec = pltpu.PrefetchScalarGridSpec(
        num_scalar_prefetch=1,
        grid=(G,),
        in_specs=[
            pl.BlockSpec((R, D), lambda g, s: (g, 0)),
            pl.BlockSpec((1, D, FF), lambda g, s: (s[g], 0, 0)),
            pl.BlockSpec((1, 1, FF), lambda g, s: (s[g], 0, 0)),
            pl.BlockSpec((1, FF, D), lambda g, s: (s[g], 0, 0)),
            pl.BlockSpec((1, 1, D), lambda g, s: (s[g], 0, 0)),
        ],
        out_specs=pl.BlockSpec((R, D), lambda g, s: (g, 0)),
    )
    return pl.pallas_call(
        _gexpert_body,
        grid_spec=grid_spec,
        out_shape=jax.ShapeDtypeStruct((S, D), jnp.float32),
    )(sp, xg, p['W1'], p['b1'][:, None, :], p['W2'], p['b2'][:, None, :])


def _final_body(y0_ref, y1_ref, route_ref, lng_ref, lnb_ref, wv_ref, bv_ref,
                wc_ref, bc_ref, sv_ref, gv_ref, cv_ref, gv_all):
    i = pl.program_id(0)
    blk = route_ref[...]
    colid = lax.broadcasted_iota(jnp.int32, (NPATCH, EPAD), 1)
    g0 = jnp.sum(jnp.where(colid == 2, blk, 0.0), axis=1, keepdims=True)
    g1 = jnp.sum(jnp.where(colid == 3, blk, 0.0), axis=1, keepdims=True)
    a = g0 * y0_ref[...] + g1 * y1_ref[...]
    mu = jnp.mean(a, axis=1, keepdims=True)
    var = jnp.mean((a - mu) ** 2, axis=1, keepdims=True)
    ln = lng_ref[...] * (a - mu) * lax.rsqrt(var + 1e-5) + lnb_ref[...]
    proj = _mm(ln, wv_ref[...]) + bv_ref[...]
    sv_ref[...] = proj
    grow = jnp.sum(proj, axis=0, keepdims=True)
    gv_ref[0] = grow
    gv_all[pl.ds(i, 1), :] = grow

    @pl.when(i == B - 1)
    def _():
        cv_ref[...] = _mm(gv_all[...], wc_ref[...]) + bc_ref[...]


def _final(y0, y1, route, p, wv, bv, wc, bc):
    full = lambda a: pl.BlockSpec(a.shape, lambda i: (0,) * a.ndim)
    args = [y0, y1, route, _row(p['ln_g']), _row(p['ln_b']), wv, _row(bv),
            wc, _row(bc)]
    return pl.pallas_call(
        _final_body,
        grid=(B,),
        in_specs=[pl.BlockSpec((NPATCH, D), lambda i: (i, 0)),
                  pl.BlockSpec((NPATCH, D), lambda i: (i, 0)),
                  pl.BlockSpec((NPATCH, EPAD), lambda i: (i, 0))]
                 + [full(a) for a in args[3:]],
        out_specs=[pl.BlockSpec((NPATCH, D), lambda i: (i, 0)),
                   pl.BlockSpec((1, 1, D), lambda i: (i, 0, 0)),
                   pl.BlockSpec((B, D), lambda i: (0, 0))],
        out_shape=[jax.ShapeDtypeStruct((NTOK, D), jnp.float32),
                   jax.ShapeDtypeStruct((B, 1, D), jnp.float32),
                   jax.ShapeDtypeStruct((B, D), jnp.float32)],
        scratch_shapes=[pltpu.VMEM((B, D), jnp.float32)],
    )(*args)


def _sparse_block(attn, pos0i, pos1i, spi, p):
    pos0 = pos0i.reshape(NTOK)
    pos1 = pos1i.reshape(NTOK)
    sp = spi.reshape(EPAD)[:G + 1]
    xg = _sc_dispatch(attn, pos0, pos1)
    eo = _gexpert(sp, xg, p)
    return _sc_combine(eo, pos0, pos1)


def kernel(x, params):
    p = params
    hp = IMG // PATCH
    patches = x.reshape(B, 1, hp, PATCH, hp, PATCH).transpose(0, 1, 2, 4, 3, 5)
    patches = patches.reshape(B, 1, hp * hp, PDIM).transpose(0, 2, 1, 3)
    patches = patches.reshape(NTOK, PDIM)
    patches = jnp.pad(patches, ((0, 0), (0, PDIM_PAD - PDIM)))
    wpe = jnp.pad(p['W_pe'], ((0, PDIM_PAD - PDIM), (0, 0)))

    m1, m2 = p['moe1'], p['moe2']
    attn1, route1, p01, p11, sp1 = _mega(
        m1, _NOISE1, True, (patches, wpe, p['b_pe'], _PE))
    y0a, y1a = _sparse_block(attn1, p01, p11, sp1, m1)

    fv, attn2, route2, p02, p12, sp2 = _mega(
        m2, _NOISE2, False,
        (y0a, y1a, route1, m1['ln_g'], m1['ln_b'], p['W_v'], p['b_v']))
    y0b, y1b = _sparse_block(attn2, p02, p12, sp2, m2)

    sv, gv, cv = _final(y0b, y1b, route2, m2, p['W_v'], p['b_v'],
                        p['W_c'], p['b_c'])

    fv = fv.reshape(B, NPATCH, D)
    sv = sv.reshape(B, NPATCH, D)
    return (fv, sv, gv.reshape(B, D), cv)
